# speedup vs baseline: 1.0628x; 1.0628x over previous
import functools

import jax
import jax.numpy as jnp
from jax import lax
from jax.experimental import pallas as pl
from jax.experimental.pallas import tpu as pltpu
from jax.experimental.pallas import tpu_sc as plsc


def _lr_sparsecore(idx_arr, flat_table, batch):
    info = plsc.get_sparse_core_info()
    nc, ns, nl = info.num_cores, info.num_subcores, info.num_lanes
    nw = nc * ns
    n_per_w = idx_arr.shape[1]
    bpw = batch // nw
    nfields = n_per_w // bpw

    mesh = plsc.VectorSubcoreMesh(core_axis_name="c", subcore_axis_name="s")

    @functools.partial(
        pl.kernel,
        mesh=mesh,
        out_type=jax.ShapeDtypeStruct((batch,), jnp.float32),
        scratch_types=[
            pltpu.VMEM((n_per_w,), jnp.int32),
            pltpu.VMEM((n_per_w,), jnp.float32),
            pltpu.VMEM((bpw,), jnp.float32),
            pltpu.SemaphoreType.DMA,
        ],
    )
    def lr_kernel(idx_hbm, table_hbm, out_hbm, idx_v, w_v, acc_v, sem):
        wid = lax.axis_index("s") * nc + lax.axis_index("c")
        pltpu.sync_copy(idx_hbm.at[wid], idx_v)
        pltpu.async_copy(table_hbm.at[idx_v], w_v, sem).wait()

        def body(k, _):
            base = k * nl
            acc = w_v[pl.ds(base, nl)]
            for f in range(1, nfields):
                acc = acc + w_v[pl.ds(f * bpw + base, nl)]
            acc_v[pl.ds(base, nl)] = acc
            return 0

        lax.fori_loop(0, bpw // nl, body, 0)
        pltpu.sync_copy(acc_v, out_hbm.at[pl.ds(wid * bpw, bpw)])

    return lr_kernel(idx_arr, flat_table)


def _fm_tensorcore(emb2d, lr_col, bias11, sel):
    batch, fd = emb2d.shape
    d = sel.shape[1]
    blk = 1024
    grid = (batch // blk,)

    def body(emb_ref, lr_ref, bias_ref, sel_ref, out_ref):
        x = emb_ref[...]
        sum_sq = jnp.sum(x * x, axis=1)
        s = jnp.dot(x, sel_ref[...],
                    preferred_element_type=jnp.float32)
        dot_sum = 0.5 * (jnp.sum(s * s, axis=1) - sum_sq)
        out_ref[...] = dot_sum[:, None] + lr_ref[...] + bias_ref[0, 0]

    return pl.pallas_call(
        body,
        grid=grid,
        in_specs=[
            pl.BlockSpec((blk, fd), lambda i: (i, 0)),
            pl.BlockSpec((blk, 1), lambda i: (i, 0)),
            pl.BlockSpec((1, 1), lambda i: (0, 0)),
            pl.BlockSpec((fd, d), lambda i: (0, 0)),
        ],
        out_specs=pl.BlockSpec((blk, 1), lambda i: (i, 0)),
        out_shape=jax.ShapeDtypeStruct((batch, 1), jnp.float32),
    )(emb2d, lr_col, bias11, sel)


def kernel(X, feature_emb, lr_table, bias):
    batch, nfields = X.shape
    vocab = lr_table.shape[1]
    d = feature_emb.shape[2]

    info = plsc.get_sparse_core_info()
    nw = info.num_cores * info.num_subcores
    bpw = batch // nw

    idx = X + jnp.arange(nfields, dtype=X.dtype)[None, :] * vocab
    idx_arr = (
        idx.reshape(nw, bpw, nfields)
        .transpose(0, 2, 1)
        .reshape(nw, nfields * bpw)
    )

    lr_vec = _lr_sparsecore(idx_arr, lr_table.reshape(-1), batch)

    sel = jnp.tile(jnp.eye(d, dtype=jnp.float32), (nfields, 1))
    out = _fm_tensorcore(
        feature_emb.reshape(batch, nfields * d),
        lr_vec[:, None],
        bias.reshape(1, 1),
        sel,
    )
    return out
